# baseline (speedup 1.0000x reference)
import jax
import jax.numpy as jnp
from jax.experimental import pallas as pl
from jax.experimental.pallas import tpu as pltpu


def _copy_body(in_ref, out_ref, sem):
    seq_len = out_ref.shape[1]
    copy = pltpu.make_async_copy(in_ref.at[pl.ds(0, seq_len)], out_ref.at[0], sem)
    copy.start()
    copy.wait()


def kernel(x, emb_table):
    seq_len = x.shape[1]
    hidden = emb_table.shape[1]
    return pl.pallas_call(
        _copy_body,
        out_shape=jax.ShapeDtypeStruct((1, seq_len, hidden), jnp.float32),
        in_specs=[pl.BlockSpec(memory_space=pl.ANY)],
        out_specs=pl.BlockSpec(memory_space=pl.ANY),
        scratch_shapes=[pltpu.SemaphoreType.DMA],
    )(emb_table)
